# baseline (speedup 1.0000x reference)
import functools

import jax
import jax.numpy as jnp
from jax import lax
from jax.experimental import pallas as pl
from jax.experimental.pallas import tpu as pltpu
from jax.experimental.pallas import tpu_sc as plsc

VOCAB = 100000
EMB = 128
BATCH = 16384

NC = 2
NS = 16
L = 16
CHUNK = 128
NACC = 4

R0 = 768
R1 = 256
NCH0 = R0 // CHUNK
NCH1 = R1 // CHUNK
assert NS * (R0 + R1) == BATCH


def _chunk_compute(t_buf, c_buf, w_t, w_c, b_s, lane, tr_buf, out_v, c_base):
    lane16 = lane * L

    def gbody(g, carry):
        r0 = g * L
        for rr in range(L):
            r = r0 + rr
            accs = [jnp.zeros((L,), jnp.float32) for _ in range(NACC)]
            for k in range(EMB // L):
                vt = t_buf[r, pl.ds(k * L, L)]
                accs[k % NACC] = accs[k % NACC] + vt * w_t[k]
            for k in range(EMB // L):
                vc = c_buf[r, pl.ds(k * L, L)]
                accs[(k + 2) % NACC] = accs[(k + 2) % NACC] + vc * w_c[k]
            part = (accs[0] + accs[1]) + (accs[2] + accs[3])
            plsc.store_scatter(tr_buf, [lane16 + rr], part)
        sums = [tr_buf[pl.ds(l * L, L)] for l in range(0, L, NACC)]
        for l in range(L):
            if l % NACC:
                sums[l // NACC] = sums[l // NACC] + tr_buf[pl.ds(l * L, L)]
        x = (sums[0] + sums[1]) + (sums[2] + sums[3]) + b_s
        out_v[pl.ds(c_base + g * L, L)] = 1.0 / (1.0 + jnp.exp(-x))
        return carry

    lax.fori_loop(0, CHUNK // L, gbody, 0, unroll=False)


def _body(t_idx_hbm, c_idx_hbm, tt_hbm, ct_hbm, wb_hbm, out_hbm,
          t_idx_v, c_idx_v, wb_v, t_buf, c_buf, tr_buf, out_v,
          sem_t0, sem_t1, sem_c0, sem_c1):
    core = lax.axis_index("c")
    sub = lax.axis_index("s")
    is0 = core == 0
    base = jnp.where(is0, sub * R0, NS * R0 + sub * R1)

    pltpu.sync_copy(wb_hbm, wb_v)

    @pl.when(is0)
    def _():
        pltpu.sync_copy(t_idx_hbm.at[pl.ds(base, R0)], t_idx_v)
        pltpu.sync_copy(c_idx_hbm.at[pl.ds(base, R0)], c_idx_v)

    @pl.when(jnp.logical_not(is0))
    def _():
        pltpu.sync_copy(t_idx_hbm.at[pl.ds(base, R1)], t_idx_v.at[pl.ds(0, R1)])
        pltpu.sync_copy(c_idx_hbm.at[pl.ds(base, R1)], c_idx_v.at[pl.ds(0, R1)])

    sems = [(sem_t0, sem_c0), (sem_t1, sem_c1)]

    def start(c):
        s = c % 2
        pltpu.async_copy(tt_hbm.at[t_idx_v.at[pl.ds(c * CHUNK, CHUNK)]],
                         t_buf.at[s], sems[s][0])
        pltpu.async_copy(ct_hbm.at[c_idx_v.at[pl.ds(c * CHUNK, CHUNK)]],
                         c_buf.at[s], sems[s][1])

    def wait(c):
        s = c % 2
        pltpu.make_async_copy(tt_hbm.at[t_idx_v.at[pl.ds(c * CHUNK, CHUNK)]],
                              t_buf.at[s], sems[s][0]).wait()
        pltpu.make_async_copy(ct_hbm.at[c_idx_v.at[pl.ds(c * CHUNK, CHUNK)]],
                              c_buf.at[s], sems[s][1]).wait()

    b_s = wb_v[pl.ds(2 * EMB, L)][0]
    w_t = [wb_v[pl.ds(k * L, L)] for k in range(EMB // L)]
    w_c = [wb_v[pl.ds(EMB + k * L, L)] for k in range(EMB // L)]
    lane = lax.iota(jnp.int32, L)

    start(0)
    for c in range(NCH0):
        nxt = c + 1
        if nxt < NCH1:
            start(nxt)
        elif nxt < NCH0:
            @pl.when(is0)
            def _(nxt=nxt):
                start(nxt)

        def do_chunk(c=c):
            wait(c)
            _chunk_compute(t_buf.at[c % 2], c_buf.at[c % 2], w_t, w_c, b_s,
                           lane, tr_buf, out_v, c * CHUNK)

        if c < NCH1:
            do_chunk()
        else:
            pl.when(is0)(do_chunk)

    @pl.when(is0)
    def _():
        pltpu.sync_copy(out_v, out_hbm.at[pl.ds(base, R0)])

    @pl.when(jnp.logical_not(is0))
    def _():
        pltpu.sync_copy(out_v.at[pl.ds(0, R1)], out_hbm.at[pl.ds(base, R1)])


def _make_kernel():
    mesh = plsc.VectorSubcoreMesh(core_axis_name="c", subcore_axis_name="s")
    return pl.kernel(
        _body,
        mesh=mesh,
        compiler_params=pltpu.CompilerParams(needs_layout_passes=False),
        out_type=jax.ShapeDtypeStruct((BATCH,), jnp.float32),
        scratch_types=[
            pltpu.VMEM((R0,), jnp.int32),
            pltpu.VMEM((R0,), jnp.int32),
            pltpu.VMEM((2 * EMB + L,), jnp.float32),
            pltpu.VMEM((2, CHUNK, EMB), jnp.float32),
            pltpu.VMEM((2, CHUNK, EMB), jnp.float32),
            pltpu.VMEM((L * L,), jnp.float32),
            pltpu.VMEM((R0,), jnp.float32),
            pltpu.SemaphoreType.DMA,
            pltpu.SemaphoreType.DMA,
            pltpu.SemaphoreType.DMA,
            pltpu.SemaphoreType.DMA,
        ],
    )


_sc_call = _make_kernel()


@jax.jit
def _run(t_idx, c_idx, target_table, context_table, wb):
    return _sc_call(t_idx, c_idx, target_table, context_table, wb)


def kernel(inputs, target_table, context_table, W, b):
    idx = inputs.astype(jnp.int32)
    t_idx = idx[:, 0]
    c_idx = idx[:, 1]
    wb = jnp.concatenate([W.reshape(-1), b,
                          jnp.zeros((L - 1,), jnp.float32)])
    out = _run(t_idx, c_idx, target_table, context_table, wb)
    return out.reshape(BATCH, 1)


# speedup vs baseline: 1.0238x; 1.0238x over previous
import functools

import jax
import jax.numpy as jnp
from jax import lax
from jax.experimental import pallas as pl
from jax.experimental.pallas import tpu as pltpu
from jax.experimental.pallas import tpu_sc as plsc

VOCAB = 100000
EMB = 128
BATCH = 16384

NC = 2
NS = 16
L = 16
CHUNK = 128
NACC = 4

R0 = 768
R1 = 256
NCH0 = R0 // CHUNK
NCH1 = R1 // CHUNK
assert NS * (R0 + R1) == BATCH


def _chunk_compute(t_buf, c_buf, w_t, w_c, b_s, lane, tr_buf, out_v, c_base):
    lane16 = lane * L

    def gbody(g, carry):
        r0 = g * L
        for rr in range(L):
            r = r0 + rr
            accs = [jnp.zeros((L,), jnp.float32) for _ in range(NACC)]
            for k in range(EMB // L):
                vt = t_buf[r, pl.ds(k * L, L)]
                accs[k % NACC] = accs[k % NACC] + vt * w_t[k]
            for k in range(EMB // L):
                vc = c_buf[r, pl.ds(k * L, L)]
                accs[(k + 2) % NACC] = accs[(k + 2) % NACC] + vc * w_c[k]
            part = (accs[0] + accs[1]) + (accs[2] + accs[3])
            plsc.store_scatter(tr_buf, [lane16 + rr], part)
        sums = [tr_buf[pl.ds(l * L, L)] for l in range(0, L, NACC)]
        for l in range(L):
            if l % NACC:
                sums[l // NACC] = sums[l // NACC] + tr_buf[pl.ds(l * L, L)]
        x = (sums[0] + sums[1]) + (sums[2] + sums[3]) + b_s
        out_v[pl.ds(c_base + g * L, L)] = 1.0 / (1.0 + jnp.exp(-x))
        return carry

    lax.fori_loop(0, CHUNK // L, gbody, 0, unroll=False)


def _body(t_idx_hbm, c_idx_hbm, tt_hbm, ct_hbm, wb_hbm, out_hbm,
          t_idx_v, c_idx_v, wb_v, t_buf, c_buf, tr_buf, out_v,
          sem_t0, sem_t1, sem_c0, sem_c1):
    core = lax.axis_index("c")
    sub = lax.axis_index("s")
    is0 = core == 1
    base = jnp.where(is0, sub * R0, NS * R0 + sub * R1)

    pltpu.sync_copy(wb_hbm, wb_v)

    @pl.when(is0)
    def _():
        pltpu.sync_copy(t_idx_hbm.at[pl.ds(base, R0)], t_idx_v)
        pltpu.sync_copy(c_idx_hbm.at[pl.ds(base, R0)], c_idx_v)

    @pl.when(jnp.logical_not(is0))
    def _():
        pltpu.sync_copy(t_idx_hbm.at[pl.ds(base, R1)], t_idx_v.at[pl.ds(0, R1)])
        pltpu.sync_copy(c_idx_hbm.at[pl.ds(base, R1)], c_idx_v.at[pl.ds(0, R1)])

    sems = [(sem_t0, sem_c0), (sem_t1, sem_c1)]

    def start(c):
        s = c % 2
        pltpu.async_copy(tt_hbm.at[t_idx_v.at[pl.ds(c * CHUNK, CHUNK)]],
                         t_buf.at[s], sems[s][0])
        pltpu.async_copy(ct_hbm.at[c_idx_v.at[pl.ds(c * CHUNK, CHUNK)]],
                         c_buf.at[s], sems[s][1])

    def wait(c):
        s = c % 2
        pltpu.make_async_copy(tt_hbm.at[t_idx_v.at[pl.ds(c * CHUNK, CHUNK)]],
                              t_buf.at[s], sems[s][0]).wait()
        pltpu.make_async_copy(ct_hbm.at[c_idx_v.at[pl.ds(c * CHUNK, CHUNK)]],
                              c_buf.at[s], sems[s][1]).wait()

    b_s = wb_v[pl.ds(2 * EMB, L)][0]
    w_t = [wb_v[pl.ds(k * L, L)] for k in range(EMB // L)]
    w_c = [wb_v[pl.ds(EMB + k * L, L)] for k in range(EMB // L)]
    lane = lax.iota(jnp.int32, L)

    start(0)
    for c in range(NCH0):
        nxt = c + 1
        if nxt < NCH1:
            start(nxt)
        elif nxt < NCH0:
            @pl.when(is0)
            def _(nxt=nxt):
                start(nxt)

        def do_chunk(c=c):
            wait(c)
            _chunk_compute(t_buf.at[c % 2], c_buf.at[c % 2], w_t, w_c, b_s,
                           lane, tr_buf, out_v, c * CHUNK)

        if c < NCH1:
            do_chunk()
        else:
            pl.when(is0)(do_chunk)

    @pl.when(is0)
    def _():
        pltpu.sync_copy(out_v, out_hbm.at[pl.ds(base, R0)])

    @pl.when(jnp.logical_not(is0))
    def _():
        pltpu.sync_copy(out_v.at[pl.ds(0, R1)], out_hbm.at[pl.ds(base, R1)])


def _make_kernel():
    mesh = plsc.VectorSubcoreMesh(core_axis_name="c", subcore_axis_name="s")
    return pl.kernel(
        _body,
        mesh=mesh,
        compiler_params=pltpu.CompilerParams(needs_layout_passes=False),
        out_type=jax.ShapeDtypeStruct((BATCH,), jnp.float32),
        scratch_types=[
            pltpu.VMEM((R0,), jnp.int32),
            pltpu.VMEM((R0,), jnp.int32),
            pltpu.VMEM((2 * EMB + L,), jnp.float32),
            pltpu.VMEM((2, CHUNK, EMB), jnp.float32),
            pltpu.VMEM((2, CHUNK, EMB), jnp.float32),
            pltpu.VMEM((L * L,), jnp.float32),
            pltpu.VMEM((R0,), jnp.float32),
            pltpu.SemaphoreType.DMA,
            pltpu.SemaphoreType.DMA,
            pltpu.SemaphoreType.DMA,
            pltpu.SemaphoreType.DMA,
        ],
    )


_sc_call = _make_kernel()


@jax.jit
def _run(t_idx, c_idx, target_table, context_table, wb):
    return _sc_call(t_idx, c_idx, target_table, context_table, wb)


def kernel(inputs, target_table, context_table, W, b):
    idx = inputs.astype(jnp.int32)
    t_idx = idx[:, 0]
    c_idx = idx[:, 1]
    wb = jnp.concatenate([W.reshape(-1), b,
                          jnp.zeros((L - 1,), jnp.float32)])
    out = _run(t_idx, c_idx, target_table, context_table, wb)
    return out.reshape(BATCH, 1)


# speedup vs baseline: 1.2877x; 1.2578x over previous
import functools

import jax
import jax.numpy as jnp
from jax import lax
from jax.experimental import pallas as pl
from jax.experimental.pallas import tpu as pltpu
from jax.experimental.pallas import tpu_sc as plsc

VOCAB = 100000
EMB = 128
BATCH = 16384

NC = 2
NS = 16
L = 16
CHUNK = 128
NACC = 4

R0 = 512
R1 = 512
NCH0 = R0 // CHUNK
NCH1 = R1 // CHUNK
assert NS * (R0 + R1) == BATCH


def _chunk_compute(t_buf, c_buf, w_t, w_c, b_s, lane, tr_buf, out_v, c_base):
    lane16 = lane * L

    def gbody(g, carry):
        r0 = g * L
        for rr in range(0, L, 2):
            ra, rb = r0 + rr, r0 + rr + 1
            accsa = [jnp.zeros((L,), jnp.float32) for _ in range(NACC)]
            accsb = [jnp.zeros((L,), jnp.float32) for _ in range(NACC)]
            for k in range(EMB // L):
                vta = t_buf[ra, pl.ds(k * L, L)]
                vtb = t_buf[rb, pl.ds(k * L, L)]
                accsa[k % NACC] = accsa[k % NACC] + vta * w_t[k]
                accsb[k % NACC] = accsb[k % NACC] + vtb * w_t[k]
            for k in range(EMB // L):
                vca = c_buf[ra, pl.ds(k * L, L)]
                vcb = c_buf[rb, pl.ds(k * L, L)]
                accsa[(k + 2) % NACC] = accsa[(k + 2) % NACC] + vca * w_c[k]
                accsb[(k + 2) % NACC] = accsb[(k + 2) % NACC] + vcb * w_c[k]
            parta = (accsa[0] + accsa[1]) + (accsa[2] + accsa[3])
            partb = (accsb[0] + accsb[1]) + (accsb[2] + accsb[3])
            plsc.store_scatter(tr_buf, [lane16 + rr], parta)
            plsc.store_scatter(tr_buf, [lane16 + rr + 1], partb)
        sums = [tr_buf[pl.ds(l * L, L)] for l in range(0, L, NACC)]
        for l in range(L):
            if l % NACC:
                sums[l // NACC] = sums[l // NACC] + tr_buf[pl.ds(l * L, L)]
        x = (sums[0] + sums[1]) + (sums[2] + sums[3]) + b_s
        out_v[pl.ds(c_base + g * L, L)] = 1.0 / (1.0 + jnp.exp(-x))
        return carry

    lax.fori_loop(0, CHUNK // L, gbody, 0, unroll=False)


def _body(t_idx_hbm, c_idx_hbm, tt_hbm, ct_hbm, wb_hbm, out_hbm,
          t_idx_v, c_idx_v, wb_v, t_buf, c_buf, tr_buf, out_v,
          sem_t0, sem_t1, sem_c0, sem_c1):
    core = lax.axis_index("c")
    sub = lax.axis_index("s")
    is0 = core == 1
    base = jnp.where(is0, sub * R0, NS * R0 + sub * R1)

    hti = pltpu.async_copy(t_idx_hbm.at[pl.ds(base, R0)], t_idx_v, sem_t0)
    hci = pltpu.async_copy(c_idx_hbm.at[pl.ds(base, R0)], c_idx_v, sem_c0)
    pltpu.sync_copy(wb_hbm, wb_v)
    hti.wait()
    hci.wait()

    sems = [(sem_t0, sem_c0), (sem_t1, sem_c1)]

    def start(c):
        s = c % 2
        pltpu.async_copy(tt_hbm.at[t_idx_v.at[pl.ds(c * CHUNK, CHUNK)]],
                         t_buf.at[s], sems[s][0])
        pltpu.async_copy(ct_hbm.at[c_idx_v.at[pl.ds(c * CHUNK, CHUNK)]],
                         c_buf.at[s], sems[s][1])

    def wait(c):
        s = c % 2
        pltpu.make_async_copy(tt_hbm.at[t_idx_v.at[pl.ds(c * CHUNK, CHUNK)]],
                              t_buf.at[s], sems[s][0]).wait()
        pltpu.make_async_copy(ct_hbm.at[c_idx_v.at[pl.ds(c * CHUNK, CHUNK)]],
                              c_buf.at[s], sems[s][1]).wait()

    b_s = wb_v[pl.ds(2 * EMB, L)][0]
    w_t = [wb_v[pl.ds(k * L, L)] for k in range(EMB // L)]
    w_c = [wb_v[pl.ds(EMB + k * L, L)] for k in range(EMB // L)]
    lane = lax.iota(jnp.int32, L)

    start(0)
    for c in range(NCH0):
        nxt = c + 1
        if nxt < NCH1:
            start(nxt)
        elif nxt < NCH0:
            @pl.when(is0)
            def _(nxt=nxt):
                start(nxt)

        def do_chunk(c=c):
            wait(c)
            _chunk_compute(t_buf.at[c % 2], c_buf.at[c % 2], w_t, w_c, b_s,
                           lane, tr_buf, out_v, c * CHUNK)

        if c < NCH1:
            do_chunk()
        else:
            pl.when(is0)(do_chunk)

    pltpu.sync_copy(out_v, out_hbm.at[pl.ds(base, R0)])


def _make_kernel():
    mesh = plsc.VectorSubcoreMesh(core_axis_name="c", subcore_axis_name="s")
    return pl.kernel(
        _body,
        mesh=mesh,
        compiler_params=pltpu.CompilerParams(needs_layout_passes=False),
        out_type=jax.ShapeDtypeStruct((BATCH,), jnp.float32),
        scratch_types=[
            pltpu.VMEM((R0,), jnp.int32),
            pltpu.VMEM((R0,), jnp.int32),
            pltpu.VMEM((2 * EMB + L,), jnp.float32),
            pltpu.VMEM((2, CHUNK, EMB), jnp.float32),
            pltpu.VMEM((2, CHUNK, EMB), jnp.float32),
            pltpu.VMEM((L * L,), jnp.float32),
            pltpu.VMEM((R0,), jnp.float32),
            pltpu.SemaphoreType.DMA,
            pltpu.SemaphoreType.DMA,
            pltpu.SemaphoreType.DMA,
            pltpu.SemaphoreType.DMA,
        ],
    )


_sc_call = _make_kernel()


@jax.jit
def _run(t_idx, c_idx, target_table, context_table, wb):
    return _sc_call(t_idx, c_idx, target_table, context_table, wb)


def kernel(inputs, target_table, context_table, W, b):
    idx = inputs.astype(jnp.int32)
    t_idx = idx[:, 0]
    c_idx = idx[:, 1]
    wb = jnp.concatenate([W.reshape(-1), b,
                          jnp.zeros((L - 1,), jnp.float32)])
    out = _run(t_idx, c_idx, target_table, context_table, wb)
    return out.reshape(BATCH, 1)


# speedup vs baseline: 1.3198x; 1.0249x over previous
import functools

import jax
import jax.numpy as jnp
from jax import lax
from jax.experimental import pallas as pl
from jax.experimental.pallas import tpu as pltpu
from jax.experimental.pallas import tpu_sc as plsc

VOCAB = 100000
EMB = 128
BATCH = 16384

NC = 2
NS = 16
L = 16
CHUNK = 128
NACC = 4

R0 = 512
R1 = 512
NCH0 = R0 // CHUNK
NCH1 = R1 // CHUNK
assert NS * (R0 + R1) == BATCH


def _chunk_compute(t_buf, c_buf, w_t, w_c, b_s, lane, tr_buf, out_v, c_base):
    lane16 = lane * L

    def gbody(g, carry):
        r0 = g * L
        nr, na = 4, 2
        for rr in range(0, L, nr):
            rows = [r0 + rr + j for j in range(nr)]
            accs = [[jnp.zeros((L,), jnp.float32) for _ in range(na)]
                    for _ in range(nr)]
            for k in range(EMB // L):
                for j, r in enumerate(rows):
                    vt = t_buf[r, pl.ds(k * L, L)]
                    accs[j][k % na] = accs[j][k % na] + vt * w_t[k]
            for k in range(EMB // L):
                for j, r in enumerate(rows):
                    vc = c_buf[r, pl.ds(k * L, L)]
                    accs[j][(k + 1) % na] = accs[j][(k + 1) % na] + vc * w_c[k]
            for j in range(nr):
                part = accs[j][0] + accs[j][1]
                plsc.store_scatter(tr_buf, [lane16 + rr + j], part)
        sums = [tr_buf[pl.ds(l * L, L)] for l in range(0, L, NACC)]
        for l in range(L):
            if l % NACC:
                sums[l // NACC] = sums[l // NACC] + tr_buf[pl.ds(l * L, L)]
        x = (sums[0] + sums[1]) + (sums[2] + sums[3]) + b_s
        out_v[pl.ds(c_base + g * L, L)] = 1.0 / (1.0 + jnp.exp(-x))
        return carry

    lax.fori_loop(0, CHUNK // L, gbody, 0, unroll=False)


def _body(t_idx_hbm, c_idx_hbm, tt_hbm, ct_hbm, wb_hbm, out_hbm,
          t_idx_v, c_idx_v, wb_v, t_buf, c_buf, tr_buf, out_v,
          sem_t0, sem_t1, sem_c0, sem_c1):
    core = lax.axis_index("c")
    sub = lax.axis_index("s")
    is0 = core == 1
    base = jnp.where(is0, sub * R0, NS * R0 + sub * R1)

    hti = pltpu.async_copy(t_idx_hbm.at[pl.ds(base, R0)], t_idx_v, sem_t0)
    hci = pltpu.async_copy(c_idx_hbm.at[pl.ds(base, R0)], c_idx_v, sem_c0)
    pltpu.sync_copy(wb_hbm, wb_v)
    hti.wait()
    hci.wait()

    sems = [(sem_t0, sem_c0), (sem_t1, sem_c1)]

    def start(c):
        s = c % 2
        pltpu.async_copy(tt_hbm.at[t_idx_v.at[pl.ds(c * CHUNK, CHUNK)]],
                         t_buf.at[s], sems[s][0])
        pltpu.async_copy(ct_hbm.at[c_idx_v.at[pl.ds(c * CHUNK, CHUNK)]],
                         c_buf.at[s], sems[s][1])

    def wait(c):
        s = c % 2
        pltpu.make_async_copy(tt_hbm.at[t_idx_v.at[pl.ds(c * CHUNK, CHUNK)]],
                              t_buf.at[s], sems[s][0]).wait()
        pltpu.make_async_copy(ct_hbm.at[c_idx_v.at[pl.ds(c * CHUNK, CHUNK)]],
                              c_buf.at[s], sems[s][1]).wait()

    b_s = wb_v[pl.ds(2 * EMB, L)][0]
    w_t = [wb_v[pl.ds(k * L, L)] for k in range(EMB // L)]
    w_c = [wb_v[pl.ds(EMB + k * L, L)] for k in range(EMB // L)]
    lane = lax.iota(jnp.int32, L)

    start(0)
    for c in range(NCH0):
        nxt = c + 1
        if nxt < NCH1:
            start(nxt)
        elif nxt < NCH0:
            @pl.when(is0)
            def _(nxt=nxt):
                start(nxt)

        def do_chunk(c=c):
            wait(c)
            _chunk_compute(t_buf.at[c % 2], c_buf.at[c % 2], w_t, w_c, b_s,
                           lane, tr_buf, out_v, c * CHUNK)

        if c < NCH1:
            do_chunk()
        else:
            pl.when(is0)(do_chunk)

    pltpu.sync_copy(out_v, out_hbm.at[pl.ds(base, R0)])


def _make_kernel():
    mesh = plsc.VectorSubcoreMesh(core_axis_name="c", subcore_axis_name="s")
    return pl.kernel(
        _body,
        mesh=mesh,
        compiler_params=pltpu.CompilerParams(needs_layout_passes=False),
        out_type=jax.ShapeDtypeStruct((BATCH,), jnp.float32),
        scratch_types=[
            pltpu.VMEM((R0,), jnp.int32),
            pltpu.VMEM((R0,), jnp.int32),
            pltpu.VMEM((2 * EMB + L,), jnp.float32),
            pltpu.VMEM((2, CHUNK, EMB), jnp.float32),
            pltpu.VMEM((2, CHUNK, EMB), jnp.float32),
            pltpu.VMEM((L * L,), jnp.float32),
            pltpu.VMEM((R0,), jnp.float32),
            pltpu.SemaphoreType.DMA,
            pltpu.SemaphoreType.DMA,
            pltpu.SemaphoreType.DMA,
            pltpu.SemaphoreType.DMA,
        ],
    )


_sc_call = _make_kernel()


@jax.jit
def _run(t_idx, c_idx, target_table, context_table, wb):
    return _sc_call(t_idx, c_idx, target_table, context_table, wb)


def kernel(inputs, target_table, context_table, W, b):
    idx = inputs.astype(jnp.int32)
    t_idx = idx[:, 0]
    c_idx = idx[:, 1]
    wb = jnp.concatenate([W.reshape(-1), b,
                          jnp.zeros((L - 1,), jnp.float32)])
    out = _run(t_idx, c_idx, target_table, context_table, wb)
    return out.reshape(BATCH, 1)


# speedup vs baseline: 1.3485x; 1.0218x over previous
import functools

import jax
import jax.numpy as jnp
from jax import lax
from jax.experimental import pallas as pl
from jax.experimental.pallas import tpu as pltpu
from jax.experimental.pallas import tpu_sc as plsc

VOCAB = 100000
EMB = 128
BATCH = 16384

NC = 2
NS = 16
L = 16
CHUNK = 128
NACC = 4

R0 = 512
R1 = 512
NCH0 = R0 // CHUNK
NCH1 = R1 // CHUNK
assert NS * (R0 + R1) == BATCH


def _chunk_compute(t_buf, c_buf, w_t, w_c, b_s, lane, tr_buf, out_v, c_base):
    lane16 = lane * L

    def gbody(g, carry):
        r0 = g * L
        nr, na = 4, 2
        for rr in range(0, L, nr):
            rows = [r0 + rr + j for j in range(nr)]
            accs = [[jnp.zeros((L,), jnp.float32) for _ in range(na)]
                    for _ in range(nr)]
            for k in range(EMB // L):
                for j, r in enumerate(rows):
                    vt = t_buf[r, pl.ds(k * L, L)]
                    accs[j][k % na] = accs[j][k % na] + vt * w_t[k]
            for k in range(EMB // L):
                for j, r in enumerate(rows):
                    vc = c_buf[r, pl.ds(k * L, L)]
                    accs[j][(k + 1) % na] = accs[j][(k + 1) % na] + vc * w_c[k]
            for j in range(nr):
                part = accs[j][0] if na == 1 else accs[j][0] + accs[j][1]
                plsc.store_scatter(tr_buf, [lane16 + rr + j], part)
        sums = [tr_buf[pl.ds(l * L, L)] for l in range(0, L, NACC)]
        for l in range(L):
            if l % NACC:
                sums[l // NACC] = sums[l // NACC] + tr_buf[pl.ds(l * L, L)]
        x = (sums[0] + sums[1]) + (sums[2] + sums[3]) + b_s
        out_v[pl.ds(c_base + g * L, L)] = 1.0 / (1.0 + jnp.exp(-x))
        return carry

    lax.fori_loop(0, CHUNK // L, gbody, 0, unroll=False)


def _body(t_idx_hbm, c_idx_hbm, tt_hbm, ct_hbm, wb_hbm, out_hbm,
          t_idx_v, c_idx_v, wb_v, t_buf, c_buf, tr_buf, out_v,
          sem_t0, sem_t1, sem_c0, sem_c1):
    core = lax.axis_index("c")
    sub = lax.axis_index("s")
    is0 = core == 1
    base = jnp.where(is0, sub * R0, NS * R0 + sub * R1)

    hti = pltpu.async_copy(t_idx_hbm.at[pl.ds(base, R0)], t_idx_v, sem_t0)
    hci = pltpu.async_copy(c_idx_hbm.at[pl.ds(base, R0)], c_idx_v, sem_c0)
    pltpu.sync_copy(wb_hbm, wb_v)
    hti.wait()
    hci.wait()

    sems = [(sem_t0, sem_c0), (sem_t1, sem_c1)]

    def start(c):
        s = c % 2
        pltpu.async_copy(tt_hbm.at[t_idx_v.at[pl.ds(c * CHUNK, CHUNK)]],
                         t_buf.at[s], sems[s][0])
        pltpu.async_copy(ct_hbm.at[c_idx_v.at[pl.ds(c * CHUNK, CHUNK)]],
                         c_buf.at[s], sems[s][1])

    def wait(c):
        s = c % 2
        pltpu.make_async_copy(tt_hbm.at[t_idx_v.at[pl.ds(c * CHUNK, CHUNK)]],
                              t_buf.at[s], sems[s][0]).wait()
        pltpu.make_async_copy(ct_hbm.at[c_idx_v.at[pl.ds(c * CHUNK, CHUNK)]],
                              c_buf.at[s], sems[s][1]).wait()

    b_s = wb_v[pl.ds(2 * EMB, L)][0]
    w_t = [wb_v[pl.ds(k * L, L)] for k in range(EMB // L)]
    w_c = [wb_v[pl.ds(EMB + k * L, L)] for k in range(EMB // L)]
    lane = lax.iota(jnp.int32, L)

    start(0)
    for c in range(NCH0):
        nxt = c + 1
        if nxt < NCH1:
            start(nxt)
        elif nxt < NCH0:
            @pl.when(is0)
            def _(nxt=nxt):
                start(nxt)

        def do_chunk(c=c):
            wait(c)
            _chunk_compute(t_buf.at[c % 2], c_buf.at[c % 2], w_t, w_c, b_s,
                           lane, tr_buf, out_v, c * CHUNK)

        if c < NCH1:
            do_chunk()
        else:
            pl.when(is0)(do_chunk)

    pltpu.sync_copy(out_v, out_hbm.at[pl.ds(base, R0)])


def _make_kernel():
    mesh = plsc.VectorSubcoreMesh(core_axis_name="c", subcore_axis_name="s")
    return pl.kernel(
        _body,
        mesh=mesh,
        compiler_params=pltpu.CompilerParams(needs_layout_passes=False,
                                             disable_bounds_checks=True,
                                             skip_device_barrier=True),
        out_type=jax.ShapeDtypeStruct((BATCH,), jnp.float32),
        scratch_types=[
            pltpu.VMEM((R0,), jnp.int32),
            pltpu.VMEM((R0,), jnp.int32),
            pltpu.VMEM((2 * EMB + L,), jnp.float32),
            pltpu.VMEM((2, CHUNK, EMB), jnp.float32),
            pltpu.VMEM((2, CHUNK, EMB), jnp.float32),
            pltpu.VMEM((L * L,), jnp.float32),
            pltpu.VMEM((R0,), jnp.float32),
            pltpu.SemaphoreType.DMA,
            pltpu.SemaphoreType.DMA,
            pltpu.SemaphoreType.DMA,
            pltpu.SemaphoreType.DMA,
        ],
    )


_sc_call = _make_kernel()


@jax.jit
def _run(t_idx, c_idx, target_table, context_table, wb):
    return _sc_call(t_idx, c_idx, target_table, context_table, wb)


def kernel(inputs, target_table, context_table, W, b):
    idx = inputs.astype(jnp.int32)
    t_idx = idx[:, 0]
    c_idx = idx[:, 1]
    wb = jnp.concatenate([W.reshape(-1), b,
                          jnp.zeros((L - 1,), jnp.float32)])
    out = _run(t_idx, c_idx, target_table, context_table, wb)
    return out.reshape(BATCH, 1)
